# speedup vs baseline: 1.5268x; 1.5268x over previous
import functools

import jax
import jax.numpy as jnp
from jax import lax
from jax.experimental import pallas as pl
from jax.experimental.pallas import tpu as pltpu
from jax.experimental.pallas import tpu_sc as plsc

_D = 128
_SEQ = 512
_NW = 32
_ROWS_PER_W = _SEQ // _NW
_WIN = _SEQ + _ROWS_PER_W - 1
_WIN_PAD = 640
_CHUNK = 128
_NCHUNK = _WIN_PAD // _CHUNK


def _sc_body(table_hbm, out_hbm, idx_v, win_v, gsem, osem):
    wid = lax.axis_index("s") * 2 + lax.axis_index("c")
    t0 = 496 - _ROWS_PER_W * wid

    lane = lax.iota(jnp.int32, 16)
    for c in range(_NCHUNK):
        for r in range(_CHUNK // 16):
            t = t0 + (c * _CHUNK + r * 16) + lane
            idx = jnp.clip(511 - t, -32, 32) + 32
            idx_v[c, pl.ds(r * 16, 16)] = idx

    gathers = []
    for c in range(_NCHUNK):
        gathers.append(
            pltpu.async_copy(
                table_hbm.at[idx_v.at[c]],
                win_v.at[pl.ds(c * _CHUNK, _CHUNK)],
                gsem,
            )
        )
    for g in gathers:
        g.wait()

    base = wid * _ROWS_PER_W
    copies = []
    for k in range(_ROWS_PER_W):
        copies.append(
            pltpu.async_copy(
                win_v.at[pl.ds(_ROWS_PER_W - 1 - k, _SEQ)],
                out_hbm.at[pl.ds((base + k) * _SEQ, _SEQ)],
                osem,
            )
        )
    for c in copies:
        c.wait()


@jax.jit
def _rel_pos_sc(table):
    mesh = plsc.VectorSubcoreMesh(core_axis_name="c", subcore_axis_name="s")
    fn = functools.partial(
        pl.kernel,
        out_type=jax.ShapeDtypeStruct((_SEQ * _SEQ, _D), jnp.float32),
        mesh=mesh,
        scratch_types=[
            pltpu.VMEM((_NCHUNK, _CHUNK), jnp.int32),
            pltpu.VMEM((_WIN_PAD, _D), jnp.float32),
            pltpu.SemaphoreType.DMA,
            pltpu.SemaphoreType.DMA,
        ],
    )(_sc_body)
    return fn(table)


def kernel(seq_len, table):
    out = _rel_pos_sc(table)
    return out.reshape(_SEQ, _SEQ, _D)


# speedup vs baseline: 7.5746x; 4.9610x over previous
import functools

import jax
import jax.numpy as jnp
from jax import lax
from jax.experimental import pallas as pl
from jax.experimental.pallas import tpu as pltpu
from jax.experimental.pallas import tpu_sc as plsc

_D = 128
_SEQ = 512
_NW = 32
_ROWS_PER_W = _SEQ // _NW
_G_PAD = 1024
_STRIPE = _G_PAD // 16


def _sc_body(table_hbm, out_hbm, idx_v, stripe_v, g_sh, gsem, osem):
    cid = lax.axis_index("c")
    sid = lax.axis_index("s")
    t0 = _STRIPE * sid
    lane = lax.iota(jnp.int32, 16)
    for r in range(_STRIPE // 16):
        t = t0 + r * 16 + lane
        idx_v[pl.ds(r * 16, 16)] = jnp.clip(511 - t, -32, 32) + 32
    pltpu.async_copy(table_hbm.at[idx_v], stripe_v, gsem).wait()
    pltpu.sync_copy(stripe_v, g_sh.at[pl.ds(t0, _STRIPE)])
    plsc.subcore_barrier()

    wid = sid * 2 + cid
    base = wid * _ROWS_PER_W
    copies = []
    for k in range(_ROWS_PER_W):
        copies.append(
            pltpu.async_copy(
                g_sh.at[pl.ds(511 - (base + k), _SEQ)],
                out_hbm.at[pl.ds((base + k) * _SEQ, _SEQ)],
                osem,
            )
        )
    for c in copies:
        c.wait()


@jax.jit
def _rel_pos_sc(table):
    mesh = plsc.VectorSubcoreMesh(core_axis_name="c", subcore_axis_name="s")
    fn = functools.partial(
        pl.kernel,
        out_type=jax.ShapeDtypeStruct((_SEQ * _SEQ, _D), jnp.float32),
        mesh=mesh,
        scratch_types=[
            pltpu.VMEM((_STRIPE,), jnp.int32),
            pltpu.VMEM((_STRIPE, _D), jnp.float32),
            pltpu.VMEM_SHARED((_G_PAD, _D), jnp.float32),
            pltpu.SemaphoreType.DMA,
            pltpu.SemaphoreType.DMA,
        ],
    )(_sc_body)
    return fn(table)


def kernel(seq_len, table):
    out = _rel_pos_sc(table)
    return out.reshape(_SEQ, _SEQ, _D)
